# baseline (speedup 1.0000x reference)
import jax
import jax.numpy as jnp
from jax.experimental import pallas as pl

_K = 128
_INT_MIN = -2147483648
_INT_MAX = 2147483647


def _mask_kernel(reps_ref, mask_ref):
    x = reps_ref[...]
    rows, n = x.shape
    b = jax.lax.bitcast_convert_type(x, jnp.int32)
    key = jnp.where(b >= 0, b, (~b) ^ jnp.int32(_INT_MIN))

    def hybrid_step(i, state):
        lo, hi, done, thr, f_lo, f_hi, side = state
        lo_f = lo.astype(jnp.float32)
        span = hi.astype(jnp.float32) + 1.0 - lo_f
        frac = f_lo / jnp.maximum(f_lo - f_hi, 1.0)
        mid_f = jnp.clip(lo_f + frac * span, -2.14e9, 2.14e9)
        mid = jnp.clip(mid_f.astype(jnp.int32), lo + 1, hi)
        cnt = jnp.sum((key >= mid).astype(jnp.int32), axis=1, keepdims=True)
        res = cnt.astype(jnp.float32) - jnp.float32(_K)
        ge = cnt >= _K
        hit = (cnt == _K) & (done < 1)
        thr = jnp.where(hit, mid, thr)
        done = jnp.where(hit, jnp.int32(1), done)
        lo = jnp.where(ge, mid, lo)
        hi = jnp.where(ge, hi, mid - 1)
        stall_hi = jnp.logical_not(ge) & (side < 0)
        stall_lo = ge & (side > 0)
        f_lo = jnp.where(ge, res, jnp.where(stall_hi, f_lo * 0.5, f_lo))
        f_hi = jnp.where(ge, jnp.where(stall_lo, f_hi * 0.5, f_hi), res)
        side = jnp.where(ge, jnp.int32(1), jnp.int32(-1))
        return lo, hi, done, thr, f_lo, f_hi, side

    lo0 = jnp.min(key, axis=1, keepdims=True)
    hi0 = jnp.max(key, axis=1, keepdims=True)
    done0 = jnp.zeros((rows, 1), jnp.int32)
    thr0 = jnp.zeros((rows, 1), jnp.int32)
    flo0 = jnp.full((rows, 1), float(n - _K), jnp.float32)
    fhi0 = jnp.full((rows, 1), float(-_K), jnp.float32)
    side0 = jnp.zeros((rows, 1), jnp.int32)
    lo, hi, done, thr, _, _, _ = jax.lax.fori_loop(
        0, 13, hybrid_step, (lo0, hi0, done0, thr0, flo0, fhi0, side0))

    def mop_cond(state):
        i, lo, hi, done = state[0], state[1], state[2], state[3]
        resolved = (done > 0) | (lo >= hi)
        return (i < 40) & jnp.logical_not(jnp.all(resolved))

    def mop_step(state):
        i, lo, hi, done, thr = state
        x_xor = lo ^ hi
        mid = (lo & hi) + (x_xor >> 1) + (x_xor & 1)
        cnt = jnp.sum((key >= mid).astype(jnp.int32), axis=1, keepdims=True)
        ge = cnt >= _K
        hit = (cnt == _K) & (done < 1)
        thr = jnp.where(hit, mid, thr)
        done = jnp.where(hit, jnp.int32(1), done)
        lo = jnp.where(ge, mid, lo)
        hi = jnp.where(ge, hi, mid - 1)
        return i + 1, lo, hi, done, thr

    _, lo, hi, done, thr = jax.lax.while_loop(
        mop_cond, mop_step, (jnp.int32(0), lo, hi, done, thr))
    done = done > 0

    def tie_path(_):
        t_exact = lo
        c_gt = jnp.sum((key > t_exact).astype(jnp.int32), axis=1,
                       keepdims=True)
        need = _K - c_gt

        def index_step(_, state):
            lo_i, hi_i = state
            mid = (lo_i & hi_i) + ((lo_i ^ hi_i) >> 1)
            iota = jax.lax.broadcasted_iota(jnp.int32, (rows, n), 1)
            sel = (key == t_exact) & (iota <= mid)
            cnt = jnp.sum(sel.astype(jnp.int32), axis=1, keepdims=True)
            ge = cnt >= need
            return jnp.where(ge, lo_i, mid + 1), jnp.where(ge, mid, hi_i)

        lo0i = jnp.zeros((rows, 1), jnp.int32)
        hi0i = jnp.full((rows, 1), n - 1, jnp.int32)
        cut, _ = jax.lax.fori_loop(0, 13, index_step, (lo0i, hi0i))
        iota = jax.lax.broadcasted_iota(jnp.int32, (rows, n), 1)
        on_tie = ((key > t_exact)
                  | ((key == t_exact) & (iota <= cut))).astype(jnp.float32)
        return jnp.where(done, (key >= thr).astype(jnp.float32), on_tie)

    on = jax.lax.cond(jnp.all(done),
                      lambda _: (key >= thr).astype(jnp.float32),
                      tie_path,
                      None)
    mask_ref[...] = on * jnp.float32(2.0) - jnp.float32(1.0)


def _overlap_kernel(b_ref, out_ref):
    k = pl.program_id(0)

    @pl.when(k == 0)
    def _init():
        out_ref[...] = jnp.zeros_like(out_ref)

    blk = (b_ref[...] > 0).astype(jnp.bfloat16)
    out_ref[...] += jax.lax.dot_general(
        blk, blk, (((1,), (1,)), ((), ())),
        preferred_element_type=jnp.float32)


def kernel(reps):
    m, n = reps.shape
    rows = 256
    mask = pl.pallas_call(
        _mask_kernel,
        grid=(m // rows,),
        in_specs=[pl.BlockSpec((rows, n), lambda i: (i, 0))],
        out_specs=pl.BlockSpec((rows, n), lambda i: (i, 0)),
        out_shape=jax.ShapeDtypeStruct((m, n), jnp.float32),
    )(reps)

    bk = 1024
    overlaps = pl.pallas_call(
        _overlap_kernel,
        grid=(n // bk,),
        in_specs=[pl.BlockSpec((m, bk), lambda k: (0, k))],
        out_specs=pl.BlockSpec((m, m), lambda k: (0, 0)),
        out_shape=jax.ShapeDtypeStruct((m, m), jnp.float32),
    )(mask)
    return (mask, overlaps)


# speedup vs baseline: 1.0089x; 1.0089x over previous
import jax
import jax.numpy as jnp
from jax.experimental import pallas as pl

_K = 128
_INT_MIN = -2147483648
_INT_MAX = 2147483647


def _mask_kernel(reps_ref, mask_ref, bin_ref):
    x = reps_ref[...]
    rows, n = x.shape
    b = jax.lax.bitcast_convert_type(x, jnp.int32)
    key = jnp.where(b >= 0, b, (~b) ^ jnp.int32(_INT_MIN))

    def hybrid_step(i, state):
        lo, hi, done, thr, f_lo, f_hi, side = state
        lo_f = lo.astype(jnp.float32)
        span = hi.astype(jnp.float32) + 1.0 - lo_f
        frac = f_lo / jnp.maximum(f_lo - f_hi, 1.0)
        mid_f = jnp.clip(lo_f + frac * span, -2.14e9, 2.14e9)
        mid = jnp.clip(mid_f.astype(jnp.int32), lo + 1, hi)
        cnt = jnp.sum((key >= mid).astype(jnp.int32), axis=1, keepdims=True)
        res = cnt.astype(jnp.float32) - jnp.float32(_K)
        ge = cnt >= _K
        hit = (cnt == _K) & (done < 1)
        thr = jnp.where(hit, mid, thr)
        done = jnp.where(hit, jnp.int32(1), done)
        lo = jnp.where(ge, mid, lo)
        hi = jnp.where(ge, hi, mid - 1)
        stall_hi = jnp.logical_not(ge) & (side < 0)
        stall_lo = ge & (side > 0)
        f_lo = jnp.where(ge, res, jnp.where(stall_hi, f_lo * 0.5, f_lo))
        f_hi = jnp.where(ge, jnp.where(stall_lo, f_hi * 0.5, f_hi), res)
        side = jnp.where(ge, jnp.int32(1), jnp.int32(-1))
        return lo, hi, done, thr, f_lo, f_hi, side

    lo0 = jnp.min(key, axis=1, keepdims=True)
    hi0 = jnp.max(key, axis=1, keepdims=True)
    done0 = jnp.zeros((rows, 1), jnp.int32)
    thr0 = jnp.zeros((rows, 1), jnp.int32)
    flo0 = jnp.full((rows, 1), float(n - _K), jnp.float32)
    fhi0 = jnp.full((rows, 1), float(-_K), jnp.float32)
    side0 = jnp.zeros((rows, 1), jnp.int32)
    lo, hi, done, thr, _, _, _ = jax.lax.fori_loop(
        0, 13, hybrid_step, (lo0, hi0, done0, thr0, flo0, fhi0, side0))

    def mop_cond(state):
        i, lo, hi, done = state[0], state[1], state[2], state[3]
        resolved = (done > 0) | (lo >= hi)
        return (i < 40) & jnp.logical_not(jnp.all(resolved))

    def mop_step(state):
        i, lo, hi, done, thr = state
        x_xor = lo ^ hi
        mid = (lo & hi) + (x_xor >> 1) + (x_xor & 1)
        cnt = jnp.sum((key >= mid).astype(jnp.int32), axis=1, keepdims=True)
        ge = cnt >= _K
        hit = (cnt == _K) & (done < 1)
        thr = jnp.where(hit, mid, thr)
        done = jnp.where(hit, jnp.int32(1), done)
        lo = jnp.where(ge, mid, lo)
        hi = jnp.where(ge, hi, mid - 1)
        return i + 1, lo, hi, done, thr

    _, lo, hi, done, thr = jax.lax.while_loop(
        mop_cond, mop_step, (jnp.int32(0), lo, hi, done, thr))
    done = done > 0

    on_clean = (key >= thr).astype(jnp.float32)

    def tie_path(_):
        t_exact = lo
        c_gt = jnp.sum((key > t_exact).astype(jnp.int32), axis=1,
                       keepdims=True)
        need = _K - c_gt

        def index_step(_, state):
            lo_i, hi_i = state
            mid = (lo_i & hi_i) + ((lo_i ^ hi_i) >> 1)
            iota = jax.lax.broadcasted_iota(jnp.int32, (rows, n), 1)
            sel = (key == t_exact) & (iota <= mid)
            cnt = jnp.sum(sel.astype(jnp.int32), axis=1, keepdims=True)
            ge = cnt >= need
            return jnp.where(ge, lo_i, mid + 1), jnp.where(ge, mid, hi_i)

        lo0i = jnp.zeros((rows, 1), jnp.int32)
        hi0i = jnp.full((rows, 1), n - 1, jnp.int32)
        cut, _ = jax.lax.fori_loop(0, 13, index_step, (lo0i, hi0i))
        iota = jax.lax.broadcasted_iota(jnp.int32, (rows, n), 1)
        on_tie = ((key > t_exact)
                  | ((key == t_exact) & (iota <= cut))).astype(jnp.float32)
        return jnp.where(done, on_clean, on_tie)

    on = jax.lax.cond(jnp.all(done),
                      lambda _: on_clean,
                      tie_path,
                      None)
    mask_ref[...] = on * jnp.float32(2.0) - jnp.float32(1.0)
    bin_ref[...] = on.astype(jnp.bfloat16)


def _overlap_kernel(b_ref, out_ref):
    k = pl.program_id(0)

    @pl.when(k == 0)
    def _init():
        out_ref[...] = jnp.zeros_like(out_ref)

    blk = b_ref[...]
    out_ref[...] += jax.lax.dot_general(
        blk, blk, (((1,), (1,)), ((), ())),
        preferred_element_type=jnp.float32)


def kernel(reps):
    m, n = reps.shape
    rows = 128
    mask, binary = pl.pallas_call(
        _mask_kernel,
        grid=(m // rows,),
        in_specs=[pl.BlockSpec((rows, n), lambda i: (i, 0))],
        out_specs=[
            pl.BlockSpec((rows, n), lambda i: (i, 0)),
            pl.BlockSpec((rows, n), lambda i: (i, 0)),
        ],
        out_shape=[
            jax.ShapeDtypeStruct((m, n), jnp.float32),
            jax.ShapeDtypeStruct((m, n), jnp.bfloat16),
        ],
    )(reps)

    bk = 2048
    overlaps = pl.pallas_call(
        _overlap_kernel,
        grid=(n // bk,),
        in_specs=[pl.BlockSpec((m, bk), lambda k: (0, k))],
        out_specs=pl.BlockSpec((m, m), lambda k: (0, 0)),
        out_shape=jax.ShapeDtypeStruct((m, m), jnp.float32),
    )(binary)
    return (mask, overlaps)


# speedup vs baseline: 1.0103x; 1.0014x over previous
import jax
import jax.numpy as jnp
from jax.experimental import pallas as pl

_K = 128
_INT_MIN = -2147483648
_INT_MAX = 2147483647


def _mask_kernel(reps_ref, mask_ref, bin_ref):
    x = reps_ref[...]
    rows, n = x.shape
    b = jax.lax.bitcast_convert_type(x, jnp.int32)
    key = jnp.where(b >= 0, b, (~b) ^ jnp.int32(_INT_MIN))

    def count_ge(mid):
        acc = jnp.zeros((rows, 128), jnp.int32)
        for c in range(0, n, 128):
            acc = acc + jnp.where(key[:, c:c + 128] >= mid, 1, 0)
        return jnp.sum(acc, axis=1, keepdims=True)

    def hybrid_step(i, state):
        lo, hi, done, thr, f_lo, f_hi, side = state
        lo_f = lo.astype(jnp.float32)
        span = hi.astype(jnp.float32) + 1.0 - lo_f
        frac = f_lo / jnp.maximum(f_lo - f_hi, 1.0)
        mid_f = jnp.clip(lo_f + frac * span, -2.14e9, 2.14e9)
        mid = jnp.clip(mid_f.astype(jnp.int32), lo + 1, hi)
        cnt = count_ge(mid)
        res = cnt.astype(jnp.float32) - jnp.float32(_K)
        ge = cnt >= _K
        hit = (cnt == _K) & (done < 1)
        thr = jnp.where(hit, mid, thr)
        done = jnp.where(hit, jnp.int32(1), done)
        lo = jnp.where(ge, mid, lo)
        hi = jnp.where(ge, hi, mid - 1)
        stall_hi = jnp.logical_not(ge) & (side < 0)
        stall_lo = ge & (side > 0)
        f_lo = jnp.where(ge, res, jnp.where(stall_hi, f_lo * 0.5, f_lo))
        f_hi = jnp.where(ge, jnp.where(stall_lo, f_hi * 0.5, f_hi), res)
        side = jnp.where(ge, jnp.int32(1), jnp.int32(-1))
        return lo, hi, done, thr, f_lo, f_hi, side

    lo0 = jnp.min(key, axis=1, keepdims=True)
    hi0 = jnp.max(key, axis=1, keepdims=True)
    done0 = jnp.zeros((rows, 1), jnp.int32)
    thr0 = jnp.zeros((rows, 1), jnp.int32)
    flo0 = jnp.full((rows, 1), float(n - _K), jnp.float32)
    fhi0 = jnp.full((rows, 1), float(-_K), jnp.float32)
    side0 = jnp.zeros((rows, 1), jnp.int32)
    lo, hi, done, thr, _, _, _ = jax.lax.fori_loop(
        0, 13, hybrid_step, (lo0, hi0, done0, thr0, flo0, fhi0, side0))

    def mop_cond(state):
        i, lo, hi, done = state[0], state[1], state[2], state[3]
        resolved = (done > 0) | (lo >= hi)
        return (i < 40) & jnp.logical_not(jnp.all(resolved))

    def mop_step(state):
        i, lo, hi, done, thr = state
        x_xor = lo ^ hi
        mid = (lo & hi) + (x_xor >> 1) + (x_xor & 1)
        cnt = count_ge(mid)
        ge = cnt >= _K
        hit = (cnt == _K) & (done < 1)
        thr = jnp.where(hit, mid, thr)
        done = jnp.where(hit, jnp.int32(1), done)
        lo = jnp.where(ge, mid, lo)
        hi = jnp.where(ge, hi, mid - 1)
        return i + 1, lo, hi, done, thr

    _, lo, hi, done, thr = jax.lax.while_loop(
        mop_cond, mop_step, (jnp.int32(0), lo, hi, done, thr))
    done = done > 0

    on_clean = (key >= thr).astype(jnp.float32)

    def tie_path(_):
        t_exact = lo
        c_gt = jnp.sum((key > t_exact).astype(jnp.int32), axis=1,
                       keepdims=True)
        need = _K - c_gt

        def index_step(_, state):
            lo_i, hi_i = state
            mid = (lo_i & hi_i) + ((lo_i ^ hi_i) >> 1)
            iota = jax.lax.broadcasted_iota(jnp.int32, (rows, n), 1)
            sel = (key == t_exact) & (iota <= mid)
            cnt = jnp.sum(sel.astype(jnp.int32), axis=1, keepdims=True)
            ge = cnt >= need
            return jnp.where(ge, lo_i, mid + 1), jnp.where(ge, mid, hi_i)

        lo0i = jnp.zeros((rows, 1), jnp.int32)
        hi0i = jnp.full((rows, 1), n - 1, jnp.int32)
        cut, _ = jax.lax.fori_loop(0, 13, index_step, (lo0i, hi0i))
        iota = jax.lax.broadcasted_iota(jnp.int32, (rows, n), 1)
        on_tie = ((key > t_exact)
                  | ((key == t_exact) & (iota <= cut))).astype(jnp.float32)
        return jnp.where(done, on_clean, on_tie)

    on = jax.lax.cond(jnp.all(done),
                      lambda _: on_clean,
                      tie_path,
                      None)
    mask_ref[...] = on * jnp.float32(2.0) - jnp.float32(1.0)
    bin_ref[...] = on.astype(jnp.bfloat16)


def _overlap_kernel(b_ref, out_ref):
    k = pl.program_id(0)

    @pl.when(k == 0)
    def _init():
        out_ref[...] = jnp.zeros_like(out_ref)

    blk = b_ref[...]
    out_ref[...] += jax.lax.dot_general(
        blk, blk, (((1,), (1,)), ((), ())),
        preferred_element_type=jnp.float32)


def kernel(reps):
    m, n = reps.shape
    rows = 128
    mask, binary = pl.pallas_call(
        _mask_kernel,
        grid=(m // rows,),
        in_specs=[pl.BlockSpec((rows, n), lambda i: (i, 0))],
        out_specs=[
            pl.BlockSpec((rows, n), lambda i: (i, 0)),
            pl.BlockSpec((rows, n), lambda i: (i, 0)),
        ],
        out_shape=[
            jax.ShapeDtypeStruct((m, n), jnp.float32),
            jax.ShapeDtypeStruct((m, n), jnp.bfloat16),
        ],
    )(reps)

    bk = 2048
    overlaps = pl.pallas_call(
        _overlap_kernel,
        grid=(n // bk,),
        in_specs=[pl.BlockSpec((m, bk), lambda k: (0, k))],
        out_specs=pl.BlockSpec((m, m), lambda k: (0, 0)),
        out_shape=jax.ShapeDtypeStruct((m, m), jnp.float32),
    )(binary)
    return (mask, overlaps)


# speedup vs baseline: 1.0138x; 1.0034x over previous
import jax
import jax.numpy as jnp
from jax.experimental import pallas as pl

_K = 128
_INT_MIN = -2147483648
_INT_MAX = 2147483647


def _mask_kernel(reps_ref, mask_ref, bin_ref):
    x = reps_ref[...]
    rows, n = x.shape
    b = jax.lax.bitcast_convert_type(x, jnp.int32)
    key = jnp.where(b >= 0, b, (~b) ^ jnp.int32(_INT_MIN))

    def count_ge(mid):
        acc = jnp.zeros((rows, 128), jnp.int32)
        for c in range(0, n, 128):
            acc = acc + jnp.where(key[:, c:c + 128] >= mid, 1, 0)
        return jnp.sum(acc, axis=1, keepdims=True)

    def hybrid_step(i, state):
        lo, hi, done, thr, f_lo, f_hi, side = state
        lo_f = lo.astype(jnp.float32)
        span = hi.astype(jnp.float32) + 1.0 - lo_f
        frac = f_lo / jnp.maximum(f_lo - f_hi, 1.0)
        mid_f = jnp.clip(lo_f + frac * span, -2.14e9, 2.14e9)
        mid = jnp.clip(mid_f.astype(jnp.int32), lo + 1, hi)
        cnt = count_ge(mid)
        res = cnt.astype(jnp.float32) - jnp.float32(_K)
        ge = cnt >= _K
        hit = (cnt == _K) & (done < 1)
        thr = jnp.where(hit, mid, thr)
        done = jnp.where(hit, jnp.int32(1), done)
        lo = jnp.where(ge, mid, lo)
        hi = jnp.where(ge, hi, mid - 1)
        stall_hi = jnp.logical_not(ge) & (side < 0)
        stall_lo = ge & (side > 0)
        f_lo = jnp.where(ge, res, jnp.where(stall_hi, f_lo * 0.5, f_lo))
        f_hi = jnp.where(ge, jnp.where(stall_lo, f_hi * 0.5, f_hi), res)
        side = jnp.where(ge, jnp.int32(1), jnp.int32(-1))
        return lo, hi, done, thr, f_lo, f_hi, side

    lo0 = jnp.min(key, axis=1, keepdims=True)
    hi0 = jnp.max(key, axis=1, keepdims=True)
    done0 = jnp.zeros((rows, 1), jnp.int32)
    thr0 = jnp.zeros((rows, 1), jnp.int32)
    flo0 = jnp.full((rows, 1), float(n - _K), jnp.float32)
    fhi0 = jnp.full((rows, 1), float(-_K), jnp.float32)
    side0 = jnp.zeros((rows, 1), jnp.int32)
    lo, hi, done, thr, _, _, _ = jax.lax.fori_loop(
        0, 13, hybrid_step, (lo0, hi0, done0, thr0, flo0, fhi0, side0))

    def mop_cond(state):
        i, lo, hi, done = state[0], state[1], state[2], state[3]
        resolved = (done > 0) | (lo >= hi)
        return (i < 40) & jnp.logical_not(jnp.all(resolved))

    def mop_step(state):
        i, lo, hi, done, thr = state
        x_xor = lo ^ hi
        mid = (lo & hi) + (x_xor >> 1) + (x_xor & 1)
        cnt = count_ge(mid)
        ge = cnt >= _K
        hit = (cnt == _K) & (done < 1)
        thr = jnp.where(hit, mid, thr)
        done = jnp.where(hit, jnp.int32(1), done)
        lo = jnp.where(ge, mid, lo)
        hi = jnp.where(ge, hi, mid - 1)
        return i + 1, lo, hi, done, thr

    _, lo, hi, done, thr = jax.lax.while_loop(
        mop_cond, mop_step, (jnp.int32(0), lo, hi, done, thr))
    done = done > 0

    on_clean = (key >= thr).astype(jnp.float32)

    def tie_path(_):
        t_exact = lo
        c_gt = jnp.sum((key > t_exact).astype(jnp.int32), axis=1,
                       keepdims=True)
        need = _K - c_gt

        def index_step(_, state):
            lo_i, hi_i = state
            mid = (lo_i & hi_i) + ((lo_i ^ hi_i) >> 1)
            iota = jax.lax.broadcasted_iota(jnp.int32, (rows, n), 1)
            sel = (key == t_exact) & (iota <= mid)
            cnt = jnp.sum(sel.astype(jnp.int32), axis=1, keepdims=True)
            ge = cnt >= need
            return jnp.where(ge, lo_i, mid + 1), jnp.where(ge, mid, hi_i)

        lo0i = jnp.zeros((rows, 1), jnp.int32)
        hi0i = jnp.full((rows, 1), n - 1, jnp.int32)
        cut, _ = jax.lax.fori_loop(0, 13, index_step, (lo0i, hi0i))
        iota = jax.lax.broadcasted_iota(jnp.int32, (rows, n), 1)
        on_tie = ((key > t_exact)
                  | ((key == t_exact) & (iota <= cut))).astype(jnp.float32)
        return jnp.where(done, on_clean, on_tie)

    on = jax.lax.cond(jnp.all(done),
                      lambda _: on_clean,
                      tie_path,
                      None)
    mask_ref[...] = on * jnp.float32(2.0) - jnp.float32(1.0)
    bin_ref[...] = on.astype(jnp.int8)


def _overlap_kernel(b_ref, out_ref):
    k = pl.program_id(0)

    @pl.when(k == 0)
    def _init():
        out_ref[...] = jnp.zeros_like(out_ref)

    blk = b_ref[...].astype(jnp.bfloat16)
    out_ref[...] += jax.lax.dot_general(
        blk, blk, (((1,), (1,)), ((), ())),
        preferred_element_type=jnp.float32)


def kernel(reps):
    m, n = reps.shape
    rows = 128
    mask, binary = pl.pallas_call(
        _mask_kernel,
        grid=(m // rows,),
        in_specs=[pl.BlockSpec((rows, n), lambda i: (i, 0))],
        out_specs=[
            pl.BlockSpec((rows, n), lambda i: (i, 0)),
            pl.BlockSpec((rows, n), lambda i: (i, 0)),
        ],
        out_shape=[
            jax.ShapeDtypeStruct((m, n), jnp.float32),
            jax.ShapeDtypeStruct((m, n), jnp.int8),
        ],
    )(reps)

    bk = 2048
    overlaps = pl.pallas_call(
        _overlap_kernel,
        grid=(n // bk,),
        in_specs=[pl.BlockSpec((m, bk), lambda k: (0, k))],
        out_specs=pl.BlockSpec((m, m), lambda k: (0, 0)),
        out_shape=jax.ShapeDtypeStruct((m, m), jnp.float32),
    )(binary)
    return (mask, overlaps)
